# baseline (speedup 1.0000x reference)
import jax
import jax.numpy as jnp
from jax import lax
from jax.experimental import pallas as pl
from jax.experimental.pallas import tpu as pltpu

N_QUADS = 16384
N_QUALS = 49152
N_ENT = 128
N_REL = 32
D = 256

QBLK = 1024
NA = N_QUALS // QBLK
TQ = 256
NT = N_QUADS // TQ
K = 1024
YW = 512


def _prelude(ent_ref, rel_ref, w1_ref, pw_ref, q2_ref,
             a_ref, b_ref, ph_ref, pr_ref, pt_ref, ts_ref):
    ent = ent_ref[...]
    rel = rel_ref[...]
    w1 = w1_ref[...]
    a_ref[...] = jnp.dot(rel, w1[:D], preferred_element_type=jnp.float32)
    b_ref[...] = jnp.dot(ent, w1[D:], preferred_element_type=jnp.float32)
    pw = pw_ref[...]
    ph_ref[...] = jnp.dot(ent, pw[:D], preferred_element_type=jnp.float32)
    pr_ref[...] = jnp.dot(rel, pw[D:2 * D], preferred_element_type=jnp.float32)
    pt_ref[...] = jnp.dot(ent, pw[2 * D:3 * D], preferred_element_type=jnp.float32)
    q2 = q2_ref[...]
    for t in range(NT + 1):
        ts_ref[0, t] = jnp.sum((q2 < t * TQ).astype(jnp.int32))


def _stage_a(qt_ref, qv_ref, qq_ref, a_ref, b_ref, b1_ref, w2_ref, b2_ref,
             aw_ref, ab_ref, av_ref, y_ref):
    t = qt_ref[0, 0, :]
    v = qv_ref[0, 0, :]
    bf = jnp.bfloat16
    oh_t = (t[:, None] == lax.broadcasted_iota(jnp.int32, (QBLK, N_REL), 1)
            ).astype(bf)
    g = jnp.dot(oh_t, a_ref[...].astype(bf), preferred_element_type=jnp.float32)
    oh_v = (v[:, None] == lax.broadcasted_iota(jnp.int32, (QBLK, N_ENT), 1)
            ).astype(bf)
    g = g + jnp.dot(oh_v, b_ref[...].astype(bf), preferred_element_type=jnp.float32)
    h = jnp.maximum(g + b1_ref[...], 0.0)
    p = jnp.dot(h.astype(bf), w2_ref[...].astype(bf),
                preferred_element_type=jnp.float32) + b2_ref[...]
    u = jnp.tanh(jnp.dot(p.astype(bf), aw_ref[...].astype(bf),
                         preferred_element_type=jnp.float32)
                 + ab_ref[...])
    s = jnp.sum(u * av_ref[...], axis=1, keepdims=True)
    e = jnp.exp(s)
    y_ref[:, :D] = p * e
    y_ref[:, D:D + 128] = jnp.broadcast_to(e, (QBLK, 128))
    y_ref[:, D + 128:] = jnp.broadcast_to(
        qq_ref[0, 0, :].astype(jnp.float32)[:, None], (QBLK, 128))


def _stage_b(ts_ref, y_hbm, h_ref, r_ref, tl_ref, ph_ref, pr_ref, pt_ref,
             pw3_ref, pb_ref, out_ref, ybuf, sem):
    tile = pl.program_id(0)
    r0 = (ts_ref[0, tile] // 8) * 8
    r1 = ts_ref[0, tile + 1]
    nch = lax.div(r1 - r0 + (K - 1), K)
    base = tile * TQ

    def chunk(i, acc):
        start = r0 + i * K
        s = pl.multiple_of(jnp.minimum(start, N_QUALS - K), 8)
        cp = pltpu.make_async_copy(y_hbm.at[pl.ds(s, K), :], ybuf, sem)
        cp.start()
        cp.wait()
        y = ybuf[...]
        idf = y[:, D + 128:]
        idcat = jnp.concatenate([idf, idf], axis=1)
        lane = lax.broadcasted_iota(jnp.int32, (K, TQ), 1) + base
        m = idcat == lane.astype(jnp.float32)
        rowi = lax.broadcasted_iota(jnp.int32, (K, TQ), 0) + s
        m = jnp.logical_and(m, rowi >= start)
        mf = m.astype(jnp.bfloat16)
        return acc + lax.dot_general(mf, y[:, :D + 128].astype(jnp.bfloat16),
                                     (((0,), (0,)), ((), ())),
                                     preferred_element_type=jnp.float32)

    acc = lax.fori_loop(0, nch, chunk,
                        jnp.zeros((TQ, D + 128), jnp.float32))
    den = acc[:, D:]
    bf = jnp.bfloat16
    qe = acc[:, :D] / (jnp.concatenate([den, den], axis=1) + 1e-9)
    tok = jnp.dot(qe.astype(bf), pw3_ref[...].astype(bf),
                  preferred_element_type=jnp.float32)
    hh = h_ref[0, 0, :]
    oh = (hh[:, None] == lax.broadcasted_iota(jnp.int32, (TQ, N_ENT), 1)
          ).astype(bf)
    tok = tok + jnp.dot(oh, ph_ref[...].astype(bf),
                        preferred_element_type=jnp.float32)
    rr = r_ref[0, 0, :]
    oh = (rr[:, None] == lax.broadcasted_iota(jnp.int32, (TQ, N_REL), 1)
          ).astype(bf)
    tok = tok + jnp.dot(oh, pr_ref[...].astype(bf),
                        preferred_element_type=jnp.float32)
    tt = tl_ref[0, 0, :]
    oh = (tt[:, None] == lax.broadcasted_iota(jnp.int32, (TQ, N_ENT), 1)
          ).astype(bf)
    tok = tok + jnp.dot(oh, pt_ref[...].astype(bf),
                        preferred_element_type=jnp.float32)
    out_ref[...] = tok + pb_ref[...]


def kernel(head_idx, rel_idx, tail_idx, qual_type_idx, qual_value_idx,
           qual_to_quad, entity_embeddings, relation_embeddings,
           mlp_w1, mlp_b1, mlp_w2, mlp_b2, attn_w, attn_b, attn_v,
           proj_w, proj_b):
    f32 = jnp.float32
    q2r = qual_to_quad.reshape(N_QUALS // 128, 128)
    prelude = pl.pallas_call(
        _prelude,
        out_shape=[
            jax.ShapeDtypeStruct((N_REL, D), f32),
            jax.ShapeDtypeStruct((N_ENT, D), f32),
            jax.ShapeDtypeStruct((N_ENT, D), f32),
            jax.ShapeDtypeStruct((N_REL, D), f32),
            jax.ShapeDtypeStruct((N_ENT, D), f32),
            jax.ShapeDtypeStruct((1, 128), jnp.int32),
        ],
        out_specs=[
            pl.BlockSpec(memory_space=pltpu.VMEM),
            pl.BlockSpec(memory_space=pltpu.VMEM),
            pl.BlockSpec(memory_space=pltpu.VMEM),
            pl.BlockSpec(memory_space=pltpu.VMEM),
            pl.BlockSpec(memory_space=pltpu.VMEM),
            pl.BlockSpec(memory_space=pltpu.SMEM),
        ],
    )
    a_tab, b_tab, ph, pr, pt, ts = prelude(
        entity_embeddings, relation_embeddings, mlp_w1, proj_w, q2r)

    qtr = qual_type_idx.reshape(NA, 1, QBLK)
    qvr = qual_value_idx.reshape(NA, 1, QBLK)
    qqr = qual_to_quad.reshape(NA, 1, QBLK)
    row = lambda x: x.reshape(1, D)
    y = pl.pallas_call(
        _stage_a,
        grid=(NA,),
        in_specs=[
            pl.BlockSpec((1, 1, QBLK), lambda i: (i, 0, 0)),
            pl.BlockSpec((1, 1, QBLK), lambda i: (i, 0, 0)),
            pl.BlockSpec((1, 1, QBLK), lambda i: (i, 0, 0)),
            pl.BlockSpec((N_REL, D), lambda i: (0, 0)),
            pl.BlockSpec((N_ENT, D), lambda i: (0, 0)),
            pl.BlockSpec((1, D), lambda i: (0, 0)),
            pl.BlockSpec((D, D), lambda i: (0, 0)),
            pl.BlockSpec((1, D), lambda i: (0, 0)),
            pl.BlockSpec((D, D), lambda i: (0, 0)),
            pl.BlockSpec((1, D), lambda i: (0, 0)),
            pl.BlockSpec((1, D), lambda i: (0, 0)),
        ],
        out_specs=pl.BlockSpec((QBLK, YW), lambda i: (i, 0)),
        out_shape=jax.ShapeDtypeStruct((N_QUALS, YW), f32),
    )(qtr, qvr, qqr, a_tab, b_tab, row(mlp_b1), mlp_w2, row(mlp_b2),
      attn_w, row(attn_b), row(attn_v))

    hr = head_idx.reshape(NT, 1, TQ)
    rr = rel_idx.reshape(NT, 1, TQ)
    tr = tail_idx.reshape(NT, 1, TQ)
    tokens = pl.pallas_call(
        _stage_b,
        grid=(NT,),
        in_specs=[
            pl.BlockSpec(memory_space=pltpu.SMEM),
            pl.BlockSpec(memory_space=pl.ANY),
            pl.BlockSpec((1, 1, TQ), lambda i: (i, 0, 0)),
            pl.BlockSpec((1, 1, TQ), lambda i: (i, 0, 0)),
            pl.BlockSpec((1, 1, TQ), lambda i: (i, 0, 0)),
            pl.BlockSpec((N_ENT, D), lambda i: (0, 0)),
            pl.BlockSpec((N_REL, D), lambda i: (0, 0)),
            pl.BlockSpec((N_ENT, D), lambda i: (0, 0)),
            pl.BlockSpec((D, D), lambda i: (0, 0)),
            pl.BlockSpec((1, D), lambda i: (0, 0)),
        ],
        out_specs=pl.BlockSpec((TQ, D), lambda i: (i, 0)),
        out_shape=jax.ShapeDtypeStruct((N_QUADS, D), f32),
        scratch_shapes=[
            pltpu.VMEM((K, YW), f32),
            pltpu.SemaphoreType.DMA,
        ],
    )(ts, y, hr, rr, tr, ph, pr, pt, proj_w[3 * D:], row(proj_b))
    return tokens


# speedup vs baseline: 1.0061x; 1.0061x over previous
import jax
import jax.numpy as jnp
from jax import lax
from jax.experimental import pallas as pl
from jax.experimental.pallas import tpu as pltpu

N_QUADS = 16384
N_QUALS = 49152
N_ENT = 128
N_REL = 32
D = 256

QBLK = 1024
NA = N_QUALS // QBLK
TQ = 256
NT = N_QUADS // TQ
K = 1024
YW = 512


def _prelude(ent_ref, rel_ref, w1_ref, pw_ref, q2_ref,
             a_ref, b_ref, ph_ref, pr_ref, pt_ref, ts_ref):
    ent = ent_ref[...]
    rel = rel_ref[...]
    w1 = w1_ref[...]
    a_ref[...] = jnp.dot(rel, w1[:D], preferred_element_type=jnp.float32)
    b_ref[...] = jnp.dot(ent, w1[D:], preferred_element_type=jnp.float32)
    pw = pw_ref[...]
    ph_ref[...] = jnp.dot(ent, pw[:D], preferred_element_type=jnp.float32)
    pr_ref[...] = jnp.dot(rel, pw[D:2 * D], preferred_element_type=jnp.float32)
    pt_ref[...] = jnp.dot(ent, pw[2 * D:3 * D], preferred_element_type=jnp.float32)
    q2 = q2_ref[...]
    for t in range(NT + 1):
        ts_ref[0, t] = jnp.sum((q2 < t * TQ).astype(jnp.int32))


def _stage_a(qt_ref, qv_ref, qq_ref, a_ref, b_ref, b1_ref, w2_ref, b2_ref,
             aw_ref, ab_ref, av_ref, y_ref):
    t = qt_ref[0, 0, :]
    v = qv_ref[0, 0, :]
    bf = jnp.bfloat16
    oh_t = (t[:, None] == lax.broadcasted_iota(jnp.int32, (QBLK, N_REL), 1)
            ).astype(bf)
    g = jnp.dot(oh_t, a_ref[...].astype(bf), preferred_element_type=jnp.float32)
    oh_v = (v[:, None] == lax.broadcasted_iota(jnp.int32, (QBLK, N_ENT), 1)
            ).astype(bf)
    g = g + jnp.dot(oh_v, b_ref[...].astype(bf), preferred_element_type=jnp.float32)
    h = jnp.maximum(g + b1_ref[...], 0.0)
    p = jnp.dot(h.astype(bf), w2_ref[...].astype(bf),
                preferred_element_type=jnp.float32) + b2_ref[...]
    u = jnp.tanh(jnp.dot(p.astype(bf), aw_ref[...].astype(bf),
                         preferred_element_type=jnp.float32)
                 + ab_ref[...])
    s = jnp.sum(u * av_ref[...], axis=1, keepdims=True)
    e = jnp.exp(s)
    y_ref[:, :D] = p * e
    y_ref[:, D:D + 128] = jnp.broadcast_to(e, (QBLK, 128))
    y_ref[:, D + 128:] = jnp.broadcast_to(
        qq_ref[0, 0, :].astype(jnp.float32)[:, None], (QBLK, 128))


def _stage_b(ts_ref, y_hbm, h_ref, r_ref, tl_ref, ph_ref, pr_ref, pt_ref,
             pw3_ref, pb_ref, out_ref, ybuf, sem):
    tile = pl.program_id(0)
    r0 = (ts_ref[0, tile] // 8) * 8
    r1 = ts_ref[0, tile + 1]
    nch = lax.div(r1 - r0 + (K - 1), K)
    base = tile * TQ

    def cpi(i, slot):
        start = r0 + i * K
        s = pl.multiple_of(jnp.minimum(start, N_QUALS - K), 8)
        return pltpu.make_async_copy(y_hbm.at[pl.ds(s, K), :],
                                     ybuf.at[slot], sem.at[slot])

    @pl.when(nch > 0)
    def _():
        cpi(0, 0).start()

    def chunk(i, acc):
        start = r0 + i * K
        s = pl.multiple_of(jnp.minimum(start, N_QUALS - K), 8)
        slot = lax.rem(i, 2)

        @pl.when(i + 1 < nch)
        def _():
            cpi(i + 1, 1 - slot).start()

        cpi(i, slot).wait()
        y = ybuf[slot]
        idf = y[:, D + 128:]
        idcat = jnp.concatenate([idf, idf], axis=1)
        lane = lax.broadcasted_iota(jnp.int32, (K, TQ), 1) + base
        m = idcat == lane.astype(jnp.float32)
        rowi = lax.broadcasted_iota(jnp.int32, (K, TQ), 0) + s
        m = jnp.logical_and(m, rowi >= start)
        mf = m.astype(jnp.bfloat16)
        return acc + lax.dot_general(mf, y[:, :D + 128].astype(jnp.bfloat16),
                                     (((0,), (0,)), ((), ())),
                                     preferred_element_type=jnp.float32)

    acc = lax.fori_loop(0, nch, chunk,
                        jnp.zeros((TQ, D + 128), jnp.float32))
    den = acc[:, D:]
    bf = jnp.bfloat16
    qe = acc[:, :D] / (jnp.concatenate([den, den], axis=1) + 1e-9)
    tok = jnp.dot(qe.astype(bf), pw3_ref[...].astype(bf),
                  preferred_element_type=jnp.float32)
    hh = h_ref[0, 0, :]
    oh = (hh[:, None] == lax.broadcasted_iota(jnp.int32, (TQ, N_ENT), 1)
          ).astype(bf)
    tok = tok + jnp.dot(oh, ph_ref[...].astype(bf),
                        preferred_element_type=jnp.float32)
    rr = r_ref[0, 0, :]
    oh = (rr[:, None] == lax.broadcasted_iota(jnp.int32, (TQ, N_REL), 1)
          ).astype(bf)
    tok = tok + jnp.dot(oh, pr_ref[...].astype(bf),
                        preferred_element_type=jnp.float32)
    tt = tl_ref[0, 0, :]
    oh = (tt[:, None] == lax.broadcasted_iota(jnp.int32, (TQ, N_ENT), 1)
          ).astype(bf)
    tok = tok + jnp.dot(oh, pt_ref[...].astype(bf),
                        preferred_element_type=jnp.float32)
    out_ref[...] = tok + pb_ref[...]


def kernel(head_idx, rel_idx, tail_idx, qual_type_idx, qual_value_idx,
           qual_to_quad, entity_embeddings, relation_embeddings,
           mlp_w1, mlp_b1, mlp_w2, mlp_b2, attn_w, attn_b, attn_v,
           proj_w, proj_b):
    f32 = jnp.float32
    q2r = qual_to_quad.reshape(N_QUALS // 128, 128)
    prelude = pl.pallas_call(
        _prelude,
        out_shape=[
            jax.ShapeDtypeStruct((N_REL, D), f32),
            jax.ShapeDtypeStruct((N_ENT, D), f32),
            jax.ShapeDtypeStruct((N_ENT, D), f32),
            jax.ShapeDtypeStruct((N_REL, D), f32),
            jax.ShapeDtypeStruct((N_ENT, D), f32),
            jax.ShapeDtypeStruct((1, 128), jnp.int32),
        ],
        out_specs=[
            pl.BlockSpec(memory_space=pltpu.VMEM),
            pl.BlockSpec(memory_space=pltpu.VMEM),
            pl.BlockSpec(memory_space=pltpu.VMEM),
            pl.BlockSpec(memory_space=pltpu.VMEM),
            pl.BlockSpec(memory_space=pltpu.VMEM),
            pl.BlockSpec(memory_space=pltpu.SMEM),
        ],
    )
    a_tab, b_tab, ph, pr, pt, ts = prelude(
        entity_embeddings, relation_embeddings, mlp_w1, proj_w, q2r)

    qtr = qual_type_idx.reshape(NA, 1, QBLK)
    qvr = qual_value_idx.reshape(NA, 1, QBLK)
    qqr = qual_to_quad.reshape(NA, 1, QBLK)
    row = lambda x: x.reshape(1, D)
    y = pl.pallas_call(
        _stage_a,
        grid=(NA,),
        in_specs=[
            pl.BlockSpec((1, 1, QBLK), lambda i: (i, 0, 0)),
            pl.BlockSpec((1, 1, QBLK), lambda i: (i, 0, 0)),
            pl.BlockSpec((1, 1, QBLK), lambda i: (i, 0, 0)),
            pl.BlockSpec((N_REL, D), lambda i: (0, 0)),
            pl.BlockSpec((N_ENT, D), lambda i: (0, 0)),
            pl.BlockSpec((1, D), lambda i: (0, 0)),
            pl.BlockSpec((D, D), lambda i: (0, 0)),
            pl.BlockSpec((1, D), lambda i: (0, 0)),
            pl.BlockSpec((D, D), lambda i: (0, 0)),
            pl.BlockSpec((1, D), lambda i: (0, 0)),
            pl.BlockSpec((1, D), lambda i: (0, 0)),
        ],
        out_specs=pl.BlockSpec((QBLK, YW), lambda i: (i, 0)),
        out_shape=jax.ShapeDtypeStruct((N_QUALS, YW), f32),
    )(qtr, qvr, qqr, a_tab, b_tab, row(mlp_b1), mlp_w2, row(mlp_b2),
      attn_w, row(attn_b), row(attn_v))

    hr = head_idx.reshape(NT, 1, TQ)
    rr = rel_idx.reshape(NT, 1, TQ)
    tr = tail_idx.reshape(NT, 1, TQ)
    tokens = pl.pallas_call(
        _stage_b,
        grid=(NT,),
        in_specs=[
            pl.BlockSpec(memory_space=pltpu.SMEM),
            pl.BlockSpec(memory_space=pl.ANY),
            pl.BlockSpec((1, 1, TQ), lambda i: (i, 0, 0)),
            pl.BlockSpec((1, 1, TQ), lambda i: (i, 0, 0)),
            pl.BlockSpec((1, 1, TQ), lambda i: (i, 0, 0)),
            pl.BlockSpec((N_ENT, D), lambda i: (0, 0)),
            pl.BlockSpec((N_REL, D), lambda i: (0, 0)),
            pl.BlockSpec((N_ENT, D), lambda i: (0, 0)),
            pl.BlockSpec((D, D), lambda i: (0, 0)),
            pl.BlockSpec((1, D), lambda i: (0, 0)),
        ],
        out_specs=pl.BlockSpec((TQ, D), lambda i: (i, 0)),
        out_shape=jax.ShapeDtypeStruct((N_QUADS, D), f32),
        scratch_shapes=[
            pltpu.VMEM((2, K, YW), f32),
            pltpu.SemaphoreType.DMA((2,)),
        ],
    )(ts, y, hr, rr, tr, ph, pr, pt, proj_w[3 * D:], row(proj_b))
    return tokens


# speedup vs baseline: 1.0522x; 1.0458x over previous
import jax
import jax.numpy as jnp
from jax import lax
from jax.experimental import pallas as pl
from jax.experimental.pallas import tpu as pltpu

N_QUADS = 16384
N_QUALS = 49152
N_ENT = 128
N_REL = 32
D = 256

QBLK = 1024
NA = N_QUALS // QBLK
TQ = 256
NT = N_QUADS // TQ
K = 1024
YW = 384


def _prelude(ent_ref, rel_ref, w1_ref, pw_ref, q2_ref,
             a_ref, b_ref, ph_ref, pr_ref, pt_ref, ts_ref):
    ent = ent_ref[...]
    rel = rel_ref[...]
    w1 = w1_ref[...]
    a_ref[...] = jnp.dot(rel, w1[:D], preferred_element_type=jnp.float32)
    b_ref[...] = jnp.dot(ent, w1[D:], preferred_element_type=jnp.float32)
    pw = pw_ref[...]
    ph_ref[...] = jnp.dot(ent, pw[:D], preferred_element_type=jnp.float32)
    pr_ref[...] = jnp.dot(rel, pw[D:2 * D], preferred_element_type=jnp.float32)
    pt_ref[...] = jnp.dot(ent, pw[2 * D:3 * D], preferred_element_type=jnp.float32)
    q2 = q2_ref[...]
    for t in range(NT + 1):
        ts_ref[0, t] = jnp.sum((q2 < t * TQ).astype(jnp.int32))


def _stage_a(qt_ref, qv_ref, a_ref, b_ref, b1_ref, w2_ref, b2_ref,
             aw_ref, ab_ref, av_ref, y_ref):
    t = qt_ref[0, 0, :]
    v = qv_ref[0, 0, :]
    bf = jnp.bfloat16
    oh_t = (t[:, None] == lax.broadcasted_iota(jnp.int32, (QBLK, N_REL), 1)
            ).astype(bf)
    g = jnp.dot(oh_t, a_ref[...].astype(bf), preferred_element_type=jnp.float32)
    oh_v = (v[:, None] == lax.broadcasted_iota(jnp.int32, (QBLK, N_ENT), 1)
            ).astype(bf)
    g = g + jnp.dot(oh_v, b_ref[...].astype(bf), preferred_element_type=jnp.float32)
    h = jnp.maximum(g + b1_ref[...], 0.0)
    p = jnp.dot(h.astype(bf), w2_ref[...].astype(bf),
                preferred_element_type=jnp.float32) + b2_ref[...]
    u = jnp.tanh(jnp.dot(p.astype(bf), aw_ref[...].astype(bf),
                         preferred_element_type=jnp.float32)
                 + ab_ref[...])
    s = jnp.sum(u * av_ref[...], axis=1, keepdims=True)
    e = jnp.exp(s)
    y_ref[:, :D] = (p * e).astype(bf)
    y_ref[:, D:] = jnp.broadcast_to(e.astype(bf), (QBLK, 128))


def _stage_b(ts_ref, y_hbm, qq_hbm, h_ref, r_ref, tl_ref, ph_ref, pr_ref,
             pt_ref, pw3_ref, pb_ref, out_ref, ybuf, qbuf, semy, semq):
    tile = pl.program_id(0)
    bf = jnp.bfloat16

    def rng(t):
        return (ts_ref[0, t] // 128) * 128, ts_ref[0, t + 1]

    r0, r1 = rng(tile)
    nch = lax.div(r1 - r0 + (K - 1), K)
    base = tile * TQ

    def cps(t0, i, slot):
        s = pl.multiple_of(jnp.minimum(t0 + i * K, N_QUALS - K), 128)
        return (pltpu.make_async_copy(y_hbm.at[pl.ds(s, K), :],
                                      ybuf.at[slot], semy.at[slot]),
                pltpu.make_async_copy(qq_hbm.at[pl.ds(s, K)],
                                      qbuf.at[slot], semq.at[slot]))

    slot0 = lax.rem(tile, 2)

    @pl.when(nch > 0)
    def _():
        for c in cps(r0, 0, slot0):
            c.start()

    def chunk(i, acc):
        start = r0 + i * K
        s = pl.multiple_of(jnp.minimum(start, N_QUALS - K), 128)
        slot = jnp.where(i == 0, slot0, 2 + lax.rem(i, 2))

        @pl.when(i + 1 < nch)
        def _():
            for c in cps(r0, i + 1, 2 + lax.rem(i + 1, 2)):
                c.start()

        for c in cps(r0, i, slot):
            c.wait()
        y = ybuf[slot]
        ids = qbuf[slot][:, None]
        lane = lax.broadcasted_iota(jnp.int32, (K, TQ), 1) + base
        m = ids == lane
        rowi = lax.broadcasted_iota(jnp.int32, (K, TQ), 0) + s
        m = jnp.logical_and(m, rowi >= start)
        mf = m.astype(bf)
        return acc + lax.dot_general(mf, y,
                                     (((0,), (0,)), ((), ())),
                                     preferred_element_type=jnp.float32)

    acc = lax.fori_loop(0, nch, chunk,
                        jnp.zeros((TQ, D + 128), jnp.float32))
    den = acc[:, D:]
    qe = acc[:, :D] / (jnp.concatenate([den, den], axis=1) + 1e-9)
    tok = jnp.dot(qe.astype(bf), pw3_ref[...].astype(bf),
                  preferred_element_type=jnp.float32)
    hh = h_ref[0, 0, :]
    oh = (hh[:, None] == lax.broadcasted_iota(jnp.int32, (TQ, N_ENT), 1)
          ).astype(bf)
    tok = tok + jnp.dot(oh, ph_ref[...].astype(bf),
                        preferred_element_type=jnp.float32)
    rr = r_ref[0, 0, :]
    oh = (rr[:, None] == lax.broadcasted_iota(jnp.int32, (TQ, N_REL), 1)
          ).astype(bf)
    tok = tok + jnp.dot(oh, pr_ref[...].astype(bf),
                        preferred_element_type=jnp.float32)
    tt = tl_ref[0, 0, :]
    oh = (tt[:, None] == lax.broadcasted_iota(jnp.int32, (TQ, N_ENT), 1)
          ).astype(bf)
    tok = tok + jnp.dot(oh, pt_ref[...].astype(bf),
                        preferred_element_type=jnp.float32)
    out_ref[...] = tok + pb_ref[...]


def kernel(head_idx, rel_idx, tail_idx, qual_type_idx, qual_value_idx,
           qual_to_quad, entity_embeddings, relation_embeddings,
           mlp_w1, mlp_b1, mlp_w2, mlp_b2, attn_w, attn_b, attn_v,
           proj_w, proj_b):
    f32 = jnp.float32
    q2r = qual_to_quad.reshape(N_QUALS // 128, 128)
    prelude = pl.pallas_call(
        _prelude,
        out_shape=[
            jax.ShapeDtypeStruct((N_REL, D), f32),
            jax.ShapeDtypeStruct((N_ENT, D), f32),
            jax.ShapeDtypeStruct((N_ENT, D), f32),
            jax.ShapeDtypeStruct((N_REL, D), f32),
            jax.ShapeDtypeStruct((N_ENT, D), f32),
            jax.ShapeDtypeStruct((1, 128), jnp.int32),
        ],
        out_specs=[
            pl.BlockSpec(memory_space=pltpu.VMEM),
            pl.BlockSpec(memory_space=pltpu.VMEM),
            pl.BlockSpec(memory_space=pltpu.VMEM),
            pl.BlockSpec(memory_space=pltpu.VMEM),
            pl.BlockSpec(memory_space=pltpu.VMEM),
            pl.BlockSpec(memory_space=pltpu.SMEM),
        ],
    )
    a_tab, b_tab, ph, pr, pt, ts = prelude(
        entity_embeddings, relation_embeddings, mlp_w1, proj_w, q2r)

    qtr = qual_type_idx.reshape(NA, 1, QBLK)
    qvr = qual_value_idx.reshape(NA, 1, QBLK)
    row = lambda x: x.reshape(1, D)
    y = pl.pallas_call(
        _stage_a,
        grid=(NA,),
        in_specs=[
            pl.BlockSpec((1, 1, QBLK), lambda i: (i, 0, 0)),
            pl.BlockSpec((1, 1, QBLK), lambda i: (i, 0, 0)),
            pl.BlockSpec((N_REL, D), lambda i: (0, 0)),
            pl.BlockSpec((N_ENT, D), lambda i: (0, 0)),
            pl.BlockSpec((1, D), lambda i: (0, 0)),
            pl.BlockSpec((D, D), lambda i: (0, 0)),
            pl.BlockSpec((1, D), lambda i: (0, 0)),
            pl.BlockSpec((D, D), lambda i: (0, 0)),
            pl.BlockSpec((1, D), lambda i: (0, 0)),
            pl.BlockSpec((1, D), lambda i: (0, 0)),
        ],
        out_specs=pl.BlockSpec((QBLK, YW), lambda i: (i, 0)),
        out_shape=jax.ShapeDtypeStruct((N_QUALS, YW), jnp.bfloat16),
    )(qtr, qvr, a_tab, b_tab, row(mlp_b1), mlp_w2, row(mlp_b2),
      attn_w, row(attn_b), row(attn_v))

    hr = head_idx.reshape(NT, 1, TQ)
    rr = rel_idx.reshape(NT, 1, TQ)
    tr = tail_idx.reshape(NT, 1, TQ)
    tokens = pl.pallas_call(
        _stage_b,
        grid=(NT,),
        in_specs=[
            pl.BlockSpec(memory_space=pltpu.SMEM),
            pl.BlockSpec(memory_space=pl.ANY),
            pl.BlockSpec(memory_space=pl.ANY),
            pl.BlockSpec((1, 1, TQ), lambda i: (i, 0, 0)),
            pl.BlockSpec((1, 1, TQ), lambda i: (i, 0, 0)),
            pl.BlockSpec((1, 1, TQ), lambda i: (i, 0, 0)),
            pl.BlockSpec((N_ENT, D), lambda i: (0, 0)),
            pl.BlockSpec((N_REL, D), lambda i: (0, 0)),
            pl.BlockSpec((N_ENT, D), lambda i: (0, 0)),
            pl.BlockSpec((D, D), lambda i: (0, 0)),
            pl.BlockSpec((1, D), lambda i: (0, 0)),
        ],
        out_specs=pl.BlockSpec((TQ, D), lambda i: (i, 0)),
        out_shape=jax.ShapeDtypeStruct((N_QUADS, D), f32),
        scratch_shapes=[
            pltpu.VMEM((4, K, YW), jnp.bfloat16),
            pltpu.VMEM((4, K), jnp.int32),
            pltpu.SemaphoreType.DMA((4,)),
            pltpu.SemaphoreType.DMA((4,)),
        ],
    )(ts, y, qual_to_quad, hr, rr, tr, ph, pr, pt, proj_w[3 * D:],
      row(proj_b))
    return tokens
